# Initial kernel scaffold; baseline (speedup 1.0000x reference)
import jax
import jax.numpy as jnp
from jax.experimental import pallas as pl


def kernel(x_vehicle, x_pickup, x_dropoff, edge_attr, node_mu, W1, b1, W2, b2, W3, b3, Wv, bv, Wp, bp, Wd, bd, Wc, bc, edge_index, node_types, batch):
    raise NotImplementedError("write your pallas kernel here")


# speedup vs baseline: 3.6475x; 3.6475x over previous
import functools

import jax
import jax.numpy as jnp
from jax import lax
from jax.experimental import pallas as pl
from jax.experimental.pallas import tpu as pltpu
from jax.experimental.pallas import tpu_sc as plsc

_N = 10000
_NPAD = 10240
_E = 320000
_PD = 128
_B = 16
_NC = 2
_NS = 16
_NT = _NC * _NS
_EPT = _E // _NT
_K = 80
_NCH = _EPT // _K
_RPT = _NPAD // _NS

_SLOPE = 0.01


def _leaky(x):
    return jnp.where(x >= 0, x, _SLOPE * x)


def _sc_segsum(table, src, dst, zeros):
    mesh = plsc.VectorSubcoreMesh(core_axis_name="c", subcore_axis_name="s")

    @functools.partial(
        pl.kernel,
        mesh=mesh,
        out_type=jax.ShapeDtypeStruct((_NC * _NPAD, _PD), jnp.float32),
        scratch_types=[
            pltpu.VMEM((_K,), jnp.int32),
            pltpu.VMEM((_K,), jnp.int32),
            pltpu.VMEM((_K, _PD), jnp.float32),
            pltpu.VMEM_SHARED((_NPAD, _PD), jnp.float32),
            pltpu.SemaphoreType.DMA,
        ],
    )
    def k(table_h, src_h, dst_h, zeros_h, out_h, src_v, dst_v, rows_v, acc_sh, sem):
        cid = lax.axis_index("c")
        sid = lax.axis_index("s")
        base = (cid * _NS + sid) * _EPT
        pltpu.sync_copy(zeros_h, acc_sh.at[pl.ds(sid * _RPT, _RPT)])
        plsc.subcore_barrier()

        def body(ci, carry):
            off = base + ci * _K
            pltpu.sync_copy(src_h.at[pl.ds(off, _K)], src_v)
            pltpu.sync_copy(dst_h.at[pl.ds(off, _K)], dst_v)
            pltpu.async_copy(table_h.at[src_v], rows_v, sem).wait()
            pltpu.sync_copy(rows_v, acc_sh.at[dst_v], add=True)
            return carry

        lax.fori_loop(0, _NCH, body, 0)
        plsc.subcore_barrier()
        pltpu.sync_copy(
            acc_sh.at[pl.ds(sid * _RPT, _RPT)],
            out_h.at[pl.ds(cid * _NPAD + sid * _RPT, _RPT)])

    return k(table, src, dst, zeros)


_EB = 4000


def _ti_body(a_ref, w_ref, b_ref, o_ref):
    x = a_ref[...] * w_ref[...] + b_ref[...]
    o_ref[...] = _leaky(x)


def _tc_ti(edge_attr, W3, b3):
    return pl.pallas_call(
        _ti_body,
        grid=(_E // _EB,),
        in_specs=[
            pl.BlockSpec((_EB, 1), lambda i: (i, 0)),
            pl.BlockSpec((1, _PD), lambda i: (0, 0)),
            pl.BlockSpec((1, _PD), lambda i: (0, 0)),
        ],
        out_specs=pl.BlockSpec((_EB, _PD), lambda i: (i, 0)),
        out_shape=jax.ShapeDtypeStruct((_E, _PD), jnp.float32),
    )(edge_attr, W3, b3.reshape(1, _PD))


_RB = 512


def _c_body(t0_ref, t1_ref, af_ref, w2_ref, wb_ref, b1_ref, b2_ref, o_ref):
    t = t0_ref[...] + t1_ref[...]
    o_ref[...] = (
        jnp.dot(t, w2_ref[...], preferred_element_type=jnp.float32)
        + jnp.dot(af_ref[...], wb_ref[...], preferred_element_type=jnp.float32)
        + b1_ref[...] + b2_ref[...])


def _tc_c(aggti, af, W2, wb, b1, b2):
    nb = _NPAD // _RB
    return pl.pallas_call(
        _c_body,
        grid=(nb,),
        in_specs=[
            pl.BlockSpec((_RB, _PD), lambda i: (i, 0)),
            pl.BlockSpec((_RB, _PD), lambda i, nb=nb: (i + nb, 0)),
            pl.BlockSpec((_RB, 16), lambda i: (i, 0)),
            pl.BlockSpec((_PD, _PD), lambda i: (0, 0)),
            pl.BlockSpec((16, _PD), lambda i: (0, 0)),
            pl.BlockSpec((1, _PD), lambda i: (0, 0)),
            pl.BlockSpec((1, _PD), lambda i: (0, 0)),
        ],
        out_specs=pl.BlockSpec((_RB, _PD), lambda i: (i, 0)),
        out_shape=jax.ShapeDtypeStruct((_NPAD, _PD), jnp.float32),
    )(aggti, aggti, af, W2, wb, b1.reshape(1, _PD), b2.reshape(1, _PD))


def _round_body(a0_ref, a1_ref, w1_ref, c_ref, o_ref):
    t = a0_ref[...] + a1_ref[...]
    x = jnp.dot(t, w1_ref[...], preferred_element_type=jnp.float32) + c_ref[...]
    o_ref[...] = _leaky(x)


def _tc_round(agg, W1, C):
    nb = _NPAD // _RB
    return pl.pallas_call(
        _round_body,
        grid=(nb,),
        in_specs=[
            pl.BlockSpec((_RB, _PD), lambda i: (i, 0)),
            pl.BlockSpec((_RB, _PD), lambda i, nb=nb: (i + nb, 0)),
            pl.BlockSpec((_PD, _PD), lambda i: (0, 0)),
            pl.BlockSpec((_RB, _PD), lambda i: (i, 0)),
        ],
        out_specs=pl.BlockSpec((_RB, _PD), lambda i: (i, 0)),
        out_shape=jax.ShapeDtypeStruct((_NPAD, _PD), jnp.float32),
    )(agg, agg, W1, C)


def _final_body(mu_ref, bf_ref, wc_ref, bc_ref, o_ref, s_sum, s_cnt):
    i = pl.program_id(0)

    @pl.when(i == 0)
    def _():
        s_sum[...] = jnp.zeros_like(s_sum)
        s_cnt[...] = jnp.zeros_like(s_cnt)

    iot = lax.broadcasted_iota(jnp.int32, (1, _B), 1).astype(jnp.float32)
    onehot = (bf_ref[...] == iot)
    onehot = onehot.astype(jnp.float32)
    dn = (((0,), (0,)), ((), ()))
    s_sum[...] += lax.dot_general(onehot, mu_ref[...], dn,
                                  preferred_element_type=jnp.float32)
    s_cnt[...] += lax.dot_general(onehot, jnp.ones((_RB, 1), jnp.float32), dn,
                                  preferred_element_type=jnp.float32)

    @pl.when(i == pl.num_programs(0) - 1)
    def _():
        g = s_sum[...] / jnp.maximum(s_cnt[...], 1.0)
        z = jnp.dot(g, wc_ref[...], preferred_element_type=jnp.float32) + bc_ref[...]
        o_ref[...] = 1.0 / (1.0 + jnp.exp(-z))


def _tc_final(mu, bf, Wc, bc):
    nb = _NPAD // _RB
    return pl.pallas_call(
        _final_body,
        grid=(nb,),
        in_specs=[
            pl.BlockSpec((_RB, _PD), lambda i: (i, 0)),
            pl.BlockSpec((_RB, 1), lambda i: (i, 0)),
            pl.BlockSpec((_PD, 1), lambda i: (0, 0)),
            pl.BlockSpec((1, 1), lambda i: (0, 0)),
        ],
        out_specs=pl.BlockSpec((_B, 1), lambda i: (0, 0)),
        out_shape=jax.ShapeDtypeStruct((_B, 1), jnp.float32),
        scratch_shapes=[
            pltpu.VMEM((_B, _PD), jnp.float32),
            pltpu.VMEM((_B, 1), jnp.float32),
        ],
    )(mu, bf, Wc, bc.reshape(1, 1))


def kernel(x_vehicle, x_pickup, x_dropoff, edge_attr, node_mu,
           W1, b1, W2, b2, W3, b3, Wv, bv, Wp, bp, Wd, bd, Wc, bc,
           edge_index, node_types, batch):
    V = x_vehicle.shape[0]
    P = x_pickup.shape[0]
    D = x_dropoff.shape[0]
    src = edge_index[0]
    dst = edge_index[1]
    zeros = jnp.zeros((_RPT, _PD), jnp.float32)

    T = _tc_ti(edge_attr, W3, b3)
    aggti = _sc_segsum(T, jnp.arange(_E, dtype=jnp.int32), dst, zeros)

    af = jnp.zeros((_NPAD, 16), jnp.float32)
    af = af.at[:V, 0:2].set(x_vehicle)
    af = af.at[V:V + P, 2:5].set(x_pickup)
    af = af.at[V + P:V + P + D, 5:7].set(x_dropoff)
    af = af.at[:V, 7].set(1.0)
    af = af.at[V:V + P, 8].set(1.0)
    af = af.at[V + P:V + P + D, 9].set(1.0)
    wb = jnp.concatenate(
        [Wv, Wp, Wd, bv[None], bp[None], bd[None],
         jnp.zeros((6, _PD), jnp.float32)], axis=0)

    C = _tc_c(aggti, af, W2, wb, b1, b2)

    mu = jnp.pad(node_mu, ((0, _NPAD - _N), (0, 0)))
    for _ in range(4):
        agg = _sc_segsum(mu, src, dst, zeros)
        mu = _tc_round(agg, W1, C)

    bf = jnp.pad(batch.astype(jnp.float32), (0, _NPAD - _N),
                 constant_values=float(_B)).reshape(_NPAD, 1)
    return _tc_final(mu, bf, Wc, bc)
